# baseline (speedup 1.0000x reference)
import functools

import jax
import jax.numpy as jnp
from jax.experimental import pallas as pl
from jax.experimental.pallas import tpu as pltpu

TOPK = 100
CHUNK = 128
CB = 2048
NEG = float("-inf")


def _score_body(k_items, u_ref, w_ref, c_ref, s_ref, cm_ref, emb_ref):
    i = pl.program_id(0)

    @pl.when(i == 0)
    def _():
        emb_ref[...] = jnp.dot(u_ref[...], w_ref[...],
                               preferred_element_type=jnp.float32)

    s = jax.lax.dot_general(emb_ref[...], c_ref[...],
                            (((1,), (1,)), ((), ())),
                            preferred_element_type=jnp.float32)
    col = i * CB + jax.lax.broadcasted_iota(jnp.int32, (1, s.shape[1]), 1)
    s = jnp.where(col >= k_items, NEG, s)
    s_ref[...] = s
    cm_ref[...] = jnp.max(s.reshape(s.shape[0], CB // CHUNK, CHUNK),
                          axis=2)[None]


def _scores_and_chunkmax(users, W, candidates):
    q, d_in = users.shape
    k_items, d = candidates.shape
    ncb = (k_items + CB - 1) // CB
    kp = ncb * CB
    cand_p = jnp.pad(candidates, ((0, kp - k_items), (0, 0)))
    return pl.pallas_call(
        functools.partial(_score_body, k_items),
        grid=(ncb,),
        in_specs=[
            pl.BlockSpec((q, d_in), lambda i: (0, 0)),
            pl.BlockSpec((d_in, d), lambda i: (0, 0)),
            pl.BlockSpec((CB, d), lambda i: (i, 0)),
        ],
        out_specs=[
            pl.BlockSpec((q, CB), lambda i: (0, i)),
            pl.BlockSpec((1, q, CB // CHUNK), lambda i: (i, 0, 0)),
        ],
        out_shape=[
            jax.ShapeDtypeStruct((q, kp), jnp.float32),
            jax.ShapeDtypeStruct((ncb, q, CB // CHUNK), jnp.float32),
        ],
        scratch_shapes=[pltpu.VMEM((q, d), jnp.float32)],
    )(users, W, cand_p)


def kernel(users, W, candidates, identifiers):
    scores, _cm = _scores_and_chunkmax(users, W, candidates)
    _, idx = jax.lax.top_k(scores, TOPK)
    return jnp.take(identifiers, idx)


# speedup vs baseline: 14.2847x; 14.2847x over previous
import functools

import jax
import jax.numpy as jnp
from jax import lax
from jax.experimental import pallas as pl
from jax.experimental.pallas import tpu as pltpu
from jax.experimental.pallas import tpu_sc as plsc

TOPK = 100
CHUNK = 128
CB = 2048
CAP = 256
NCID = 128
NC, NS = 2, 16
NW = NC * NS
NEG = float("-inf")


def _v16(x):
    return jnp.full((16,), x, jnp.int32)


def _cumsum16(mask):
    x = jnp.where(mask, _v16(1), _v16(0))
    idx = lax.iota(jnp.int32, 16)
    for sh in (1, 2, 4, 8):
        src = jnp.maximum(idx - _v16(sh), _v16(0))
        shifted = lax.gather(
            x, src[:, None],
            lax.GatherDimensionNumbers(offset_dims=(),
                                       collapsed_slice_dims=(0,),
                                       start_index_map=(0,)),
            (1,), mode=lax.GatherScatterMode.PROMISE_IN_BOUNDS)
        x = x + jnp.where(idx >= _v16(sh), shifted, _v16(0))
    return x


def _score_body(k_items, u_ref, w_ref, c_ref, s_ref, cm_ref, emb_ref):
    i = pl.program_id(0)

    @pl.when(i == 0)
    def _():
        emb_ref[...] = jnp.dot(u_ref[...], w_ref[...],
                               preferred_element_type=jnp.float32)

    s = lax.dot_general(emb_ref[...], c_ref[...],
                        (((1,), (1,)), ((), ())),
                        preferred_element_type=jnp.float32)
    col = i * CB + lax.broadcasted_iota(jnp.int32, (1, s.shape[1]), 1)
    s = jnp.where(col >= k_items, NEG, s)
    s_ref[...] = s
    cm_ref[...] = jnp.max(s.reshape(s.shape[0], CB // CHUNK, CHUNK),
                          axis=2)[None]


def _scores_and_chunkmax(users, W, candidates):
    q, d_in = users.shape
    k_items, d = candidates.shape
    ncb = (k_items + CB - 1) // CB
    kp = ncb * CB
    cand_p = jnp.pad(candidates, ((0, kp - k_items), (0, 0)))
    return pl.pallas_call(
        functools.partial(_score_body, k_items),
        grid=(ncb,),
        in_specs=[
            pl.BlockSpec((q, d_in), lambda i: (0, 0)),
            pl.BlockSpec((d_in, d), lambda i: (0, 0)),
            pl.BlockSpec((CB, d), lambda i: (i, 0)),
        ],
        out_specs=[
            pl.BlockSpec((q, CB), lambda i: (0, i)),
            pl.BlockSpec((1, q, CB // CHUNK), lambda i: (i, 0, 0)),
        ],
        out_shape=[
            jax.ShapeDtypeStruct((q, kp), jnp.float32),
            jax.ShapeDtypeStruct((ncb, q, CB // CHUNK), jnp.float32),
        ],
        scratch_shapes=[pltpu.VMEM((q, d), jnp.float32)],
    )(users, W, cand_p)


def _thresh_body(cm_ref, thr_ref):
    def step(_, c):
        m = jnp.max(c, axis=1, keepdims=True)
        return jnp.where(c >= m, NEG, c)

    cm = lax.fori_loop(0, TOPK - 1, step, cm_ref[...])
    m = jnp.max(cm, axis=1, keepdims=True)
    thr_ref[...] = jnp.broadcast_to(m, (m.shape[0], 16))


def _thresholds(cm):
    q = cm.shape[0]
    return pl.pallas_call(
        _thresh_body,
        out_shape=jax.ShapeDtypeStruct((q, 16), jnp.float32),
    )(cm)


def _sc_body(nq, nchunk, s3, cmf, thr, ids, vals_o, gidx_o, idn_o,
             cm_v, t_v, cid_v, ridx_v, rows_v, val_v, gix_v, idn_v, sem):
    c = lax.axis_index("c")
    s = lax.axis_index("s")
    wid = s * NC + c
    qbase = wid * nq

    def per_query(qi, carry):
        q = qbase + qi
        pltpu.sync_copy(cmf.at[pl.ds(q * nchunk, nchunk)], cm_v)
        pltpu.sync_copy(thr.at[pl.ds(q * 16, 16)], t_v)
        t = t_v[...]

        for j in range(NCID // 16 + 1):
            cid_v[pl.ds(j * 16, 16)] = jnp.full((16,), nchunk - 1, jnp.int32)
        base = jnp.zeros((16,), jnp.int32)
        for j in range(nchunk // 16):
            cm16 = cm_v[pl.ds(j * 16, 16)]
            mask = cm16 >= t
            pos = base + _cumsum16(mask) - _v16(1)
            pos = jnp.minimum(jnp.maximum(pos, _v16(0)), _v16(NCID - 1))
            cids = _v16(j * 16) + lax.iota(jnp.int32, 16)
            plsc.store_scatter(cid_v, [pos], cids, mask=mask)
            base = base + plsc.all_reduce_population_count(mask)

        for j in range(NCID // 16):
            ridx_v[pl.ds(j * 16, 16)] = (cid_v[pl.ds(j * 16, 16)]
                                         + _v16(q * nchunk))
        pltpu.async_copy(s3.at[ridx_v], rows_v, sem).wait()

        for j in range(CAP // 16):
            val_v[pl.ds(j * 16, 16)] = jnp.full((16,), NEG, jnp.float32)
            gix_v[pl.ds(j * 16, 16)] = jnp.zeros((16,), jnp.int32)

        def per_row(r, b):
            gb = _v16(cid_v[pl.ds(r, 16)][0] * CHUNK)
            for j in range(CHUNK // 16):
                rv = rows_v[r, pl.ds(j * 16, 16)]
                mask = rv >= t
                pos = b + _cumsum16(mask) - _v16(1)
                pos = jnp.minimum(jnp.maximum(pos, _v16(0)), _v16(CAP - 1))
                g = gb + _v16(j * 16) + lax.iota(jnp.int32, 16)
                plsc.store_scatter(val_v, [pos], rv, mask=mask)
                plsc.store_scatter(gix_v, [pos], g, mask=mask)
                b = b + plsc.all_reduce_population_count(mask)
            return b

        lax.fori_loop(0, NCID, per_row, jnp.zeros((16,), jnp.int32))

        pltpu.async_copy(ids.at[gix_v.at[pl.ds(0, 128)]],
                         idn_v.at[pl.ds(0, 128)], sem).wait()
        pltpu.async_copy(ids.at[gix_v.at[pl.ds(128, 128)]],
                         idn_v.at[pl.ds(128, 128)], sem).wait()

        pltpu.sync_copy(val_v, vals_o.at[pl.ds(q * CAP, CAP)])
        pltpu.sync_copy(gix_v, gidx_o.at[pl.ds(q * CAP, CAP)])
        pltpu.sync_copy(idn_v, idn_o.at[pl.ds(q * CAP, CAP)])
        return carry

    lax.fori_loop(0, nq, per_query, 0)


def _sc_compact(scores, cm, thr, identifiers):
    q, kp = scores.shape
    nchunk = kp // CHUNK
    nq = q // NW
    s3 = scores.reshape(q * nchunk, CHUNK)
    mesh = plsc.VectorSubcoreMesh(core_axis_name="c", subcore_axis_name="s")
    f = pl.kernel(
        functools.partial(_sc_body, nq, nchunk),
        mesh=mesh,
        compiler_params=pltpu.CompilerParams(needs_layout_passes=False),
        out_type=[
            jax.ShapeDtypeStruct((q * CAP,), jnp.float32),
            jax.ShapeDtypeStruct((q * CAP,), jnp.int32),
            jax.ShapeDtypeStruct((q * CAP,), jnp.int32),
        ],
        scratch_types=[
            pltpu.VMEM((nchunk,), jnp.float32),
            pltpu.VMEM((16,), jnp.float32),
            pltpu.VMEM((NCID + 16,), jnp.int32),
            pltpu.VMEM((NCID,), jnp.int32),
            pltpu.VMEM((NCID, CHUNK), jnp.float32),
            pltpu.VMEM((CAP,), jnp.float32),
            pltpu.VMEM((CAP,), jnp.int32),
            pltpu.VMEM((CAP,), jnp.int32),
            pltpu.SemaphoreType.DMA,
        ],
    )
    vals, gidx, idn = f(s3, cm.reshape(-1), thr.reshape(-1), identifiers)
    return (vals.reshape(q, CAP), gidx.reshape(q, CAP), idn.reshape(q, CAP))


def _select_body(v_ref, gi_ref, id_ref, out_ref):
    v = v_ref[...]
    gi = gi_ref[...]
    ident = id_ref[...]
    q = v.shape[0]
    lanes = lax.broadcasted_iota(jnp.int32, (q, 128), 1)
    intmax = jnp.int32(2**31 - 1)

    def step(k, carry):
        v, out = carry
        m = jnp.max(v, axis=1, keepdims=True)
        sel = v >= m
        chosen = jnp.min(jnp.where(sel, gi, intmax), axis=1, keepdims=True)
        hit = sel & (gi == chosen)
        cid = jnp.sum(jnp.where(hit, ident, 0), axis=1, keepdims=True)
        out = out + jnp.where(lanes == k, cid, 0)
        return jnp.where(hit, NEG, v), out

    _, out = lax.fori_loop(0, TOPK, step,
                           (v, jnp.zeros((q, 128), jnp.int32)))
    out_ref[...] = out


def _select(vals, gidx, idn):
    q = vals.shape[0]
    return pl.pallas_call(
        _select_body,
        out_shape=jax.ShapeDtypeStruct((q, 128), jnp.int32),
    )(vals, gidx, idn)


def kernel(users, W, candidates, identifiers):
    q = users.shape[0]
    scores, cm3 = _scores_and_chunkmax(users, W, candidates)
    nchunk = scores.shape[1] // CHUNK
    cm = cm3.transpose(1, 0, 2).reshape(q, nchunk)
    thr = _thresholds(cm)
    vals, gidx, idn = _sc_compact(scores, cm, thr, identifiers)
    out = _select(vals, gidx, idn)
    return out[:, :TOPK]


# speedup vs baseline: 15.6143x; 1.0931x over previous
import functools

import jax
import jax.numpy as jnp
from jax import lax
from jax.experimental import pallas as pl
from jax.experimental.pallas import tpu as pltpu
from jax.experimental.pallas import tpu_sc as plsc

TOPK = 100
CHUNK = 128
CB = 2048
CAP = 256
NCID = 128
NC, NS = 2, 16
NW = NC * NS
NEG = float("-inf")


def _v16(x):
    return jnp.full((16,), x, jnp.int32)


def _score_body(k_items, u_ref, w_ref, c_ref, s_ref, cm_ref, emb_ref):
    i = pl.program_id(0)

    @pl.when(i == 0)
    def _():
        emb_ref[...] = jnp.dot(u_ref[...], w_ref[...],
                               preferred_element_type=jnp.float32)

    s = lax.dot_general(emb_ref[...], c_ref[...],
                        (((1,), (1,)), ((), ())),
                        preferred_element_type=jnp.float32)
    col = i * CB + lax.broadcasted_iota(jnp.int32, (1, s.shape[1]), 1)
    s = jnp.where(col >= k_items, NEG, s)
    s_ref[...] = s
    cm_ref[...] = jnp.max(s.reshape(s.shape[0], CB // CHUNK, CHUNK),
                          axis=2)[None]


def _scores_and_chunkmax(users, W, candidates):
    q, d_in = users.shape
    k_items, d = candidates.shape
    ncb = (k_items + CB - 1) // CB
    kp = ncb * CB
    cand_p = jnp.pad(candidates, ((0, kp - k_items), (0, 0)))
    return pl.pallas_call(
        functools.partial(_score_body, k_items),
        grid=(ncb,),
        in_specs=[
            pl.BlockSpec((q, d_in), lambda i: (0, 0)),
            pl.BlockSpec((d_in, d), lambda i: (0, 0)),
            pl.BlockSpec((CB, d), lambda i: (i, 0)),
        ],
        out_specs=[
            pl.BlockSpec((q, CB), lambda i: (0, i)),
            pl.BlockSpec((1, q, CB // CHUNK), lambda i: (i, 0, 0)),
        ],
        out_shape=[
            jax.ShapeDtypeStruct((q, kp), jnp.float32),
            jax.ShapeDtypeStruct((ncb, q, CB // CHUNK), jnp.float32),
        ],
        scratch_shapes=[pltpu.VMEM((q, d), jnp.float32)],
    )(users, W, cand_p)


def _thresh_body(cm_ref, thr_ref):
    def step(_, c):
        m = jnp.max(c, axis=1, keepdims=True)
        return jnp.where(c >= m, NEG, c)

    cm = lax.fori_loop(0, TOPK - 1, step, cm_ref[...])
    m = jnp.max(cm, axis=1, keepdims=True)
    thr_ref[...] = jnp.broadcast_to(m, (m.shape[0], 16))


def _thresholds(cm):
    q = cm.shape[0]
    return pl.pallas_call(
        _thresh_body,
        out_shape=jax.ShapeDtypeStruct((q, 16), jnp.float32),
    )(cm)


def _sc_body(nq, nchunk, s3, cmf, thr, ids, vals_o, gidx_o, idn_o,
             cm_v, t_v, cid_v, ridx_v, rows_v, val_v, gix_v, idn_v, cnt_v,
             sem):
    c = lax.axis_index("c")
    s = lax.axis_index("s")
    wid = s * NC + c
    qbase = wid * nq

    def per_query(qi, carry):
        q = qbase + qi
        pltpu.sync_copy(cmf.at[pl.ds(q * nchunk, nchunk)], cm_v)
        pltpu.sync_copy(thr.at[pl.ds(q * 16, 16)], t_v)
        t = t_v[...]

        for j in range(NCID // 16 + 1):
            cid_v[pl.ds(j * 16, 16)] = jnp.full((16,), nchunk - 1, jnp.int32)
        base = jnp.zeros((16,), jnp.int32)
        for j in range(nchunk // 16):
            cm16 = cm_v[pl.ds(j * 16, 16)]
            mask = cm16 >= t
            pos = base + plsc.cumsum(_v16(1), mask=mask) - _v16(1)
            pos = jnp.minimum(jnp.maximum(pos, _v16(0)), _v16(NCID - 1))
            cids = _v16(j * 16) + lax.iota(jnp.int32, 16)
            plsc.store_scatter(cid_v, [pos], cids, mask=mask)
            base = base + plsc.all_reduce_population_count(mask)

        cnt_v[pl.ds(0, 16)] = base
        n = jnp.minimum(cnt_v[pl.ds(0, 16)][0], NCID)

        for j in range(NCID // 16):
            ridx_v[pl.ds(j * 16, 16)] = (cid_v[pl.ds(j * 16, 16)]
                                         + _v16(q * nchunk))
        pltpu.async_copy(s3.at[ridx_v], rows_v, sem).wait()

        for j in range(CAP // 16):
            val_v[pl.ds(j * 16, 16)] = jnp.full((16,), NEG, jnp.float32)
            gix_v[pl.ds(j * 16, 16)] = jnp.zeros((16,), jnp.int32)

        def per_row(r, b):
            gb = _v16(cid_v[pl.ds(r, 16)][0] * CHUNK)
            for j in range(CHUNK // 16):
                rv = rows_v[r, pl.ds(j * 16, 16)]
                mask = rv >= t
                pos = b + plsc.cumsum(_v16(1), mask=mask) - _v16(1)
                pos = jnp.minimum(jnp.maximum(pos, _v16(0)), _v16(CAP - 1))
                g = gb + _v16(j * 16) + lax.iota(jnp.int32, 16)
                plsc.store_scatter(val_v, [pos], rv, mask=mask)
                plsc.store_scatter(gix_v, [pos], g, mask=mask)
                b = b + plsc.all_reduce_population_count(mask)
            return b

        lax.fori_loop(0, n, per_row, jnp.zeros((16,), jnp.int32))

        pltpu.async_copy(ids.at[gix_v.at[pl.ds(0, 128)]],
                         idn_v.at[pl.ds(0, 128)], sem).wait()
        pltpu.async_copy(ids.at[gix_v.at[pl.ds(128, 128)]],
                         idn_v.at[pl.ds(128, 128)], sem).wait()

        pltpu.sync_copy(val_v, vals_o.at[pl.ds(q * CAP, CAP)])
        pltpu.sync_copy(gix_v, gidx_o.at[pl.ds(q * CAP, CAP)])
        pltpu.sync_copy(idn_v, idn_o.at[pl.ds(q * CAP, CAP)])
        return carry

    lax.fori_loop(0, nq, per_query, 0)


def _sc_compact(scores, cm, thr, identifiers):
    q, kp = scores.shape
    nchunk = kp // CHUNK
    nq = q // NW
    s3 = scores.reshape(q * nchunk, CHUNK)
    mesh = plsc.VectorSubcoreMesh(core_axis_name="c", subcore_axis_name="s")
    f = pl.kernel(
        functools.partial(_sc_body, nq, nchunk),
        mesh=mesh,
        compiler_params=pltpu.CompilerParams(needs_layout_passes=False),
        out_type=[
            jax.ShapeDtypeStruct((q * CAP,), jnp.float32),
            jax.ShapeDtypeStruct((q * CAP,), jnp.int32),
            jax.ShapeDtypeStruct((q * CAP,), jnp.int32),
        ],
        scratch_types=[
            pltpu.VMEM((nchunk,), jnp.float32),
            pltpu.VMEM((16,), jnp.float32),
            pltpu.VMEM((NCID + 16,), jnp.int32),
            pltpu.VMEM((NCID,), jnp.int32),
            pltpu.VMEM((NCID, CHUNK), jnp.float32),
            pltpu.VMEM((CAP,), jnp.float32),
            pltpu.VMEM((CAP,), jnp.int32),
            pltpu.VMEM((CAP,), jnp.int32),
            pltpu.VMEM((16,), jnp.int32),
            pltpu.SemaphoreType.DMA,
        ],
    )
    vals, gidx, idn = f(s3, cm.reshape(-1), thr.reshape(-1), identifiers)
    return (vals.reshape(q, CAP), gidx.reshape(q, CAP), idn.reshape(q, CAP))


def _select_body(v_ref, gi_ref, id_ref, out_ref):
    v = v_ref[...]
    gi = gi_ref[...]
    ident = id_ref[...]
    q = v.shape[0]
    lanes = lax.broadcasted_iota(jnp.int32, (q, 128), 1)
    intmax = jnp.int32(2**31 - 1)

    def step(k, carry):
        v, out = carry
        m = jnp.max(v, axis=1, keepdims=True)
        sel = v >= m
        chosen = jnp.min(jnp.where(sel, gi, intmax), axis=1, keepdims=True)
        hit = sel & (gi == chosen)
        cid = jnp.sum(jnp.where(hit, ident, 0), axis=1, keepdims=True)
        out = out + jnp.where(lanes == k, cid, 0)
        return jnp.where(hit, NEG, v), out

    _, out = lax.fori_loop(0, TOPK, step,
                           (v, jnp.zeros((q, 128), jnp.int32)))
    out_ref[...] = out


def _select(vals, gidx, idn):
    q = vals.shape[0]
    return pl.pallas_call(
        _select_body,
        out_shape=jax.ShapeDtypeStruct((q, 128), jnp.int32),
    )(vals, gidx, idn)


def kernel(users, W, candidates, identifiers):
    q = users.shape[0]
    scores, cm3 = _scores_and_chunkmax(users, W, candidates)
    nchunk = scores.shape[1] // CHUNK
    cm = cm3.transpose(1, 0, 2).reshape(q, nchunk)
    thr = _thresholds(cm)
    vals, gidx, idn = _sc_compact(scores, cm, thr, identifiers)
    out = _select(vals, gidx, idn)
    return out[:, :TOPK]


# speedup vs baseline: 16.0930x; 1.0307x over previous
import functools

import jax
import jax.numpy as jnp
from jax import lax
from jax.experimental import pallas as pl
from jax.experimental.pallas import tpu as pltpu
from jax.experimental.pallas import tpu_sc as plsc

TOPK = 100
CHUNK = 128
CB = 2048
CAP = 256
NCID = 128
NC, NS = 2, 16
NW = NC * NS
NEG = float("-inf")


def _v16(x):
    return jnp.full((16,), x, jnp.int32)


def _score_body(k_items, u_ref, w_ref, c_ref, s_ref, cm_ref, emb_ref):
    i = pl.program_id(0)

    @pl.when(i == 0)
    def _():
        emb_ref[...] = jnp.dot(u_ref[...], w_ref[...],
                               preferred_element_type=jnp.float32)

    s = lax.dot_general(emb_ref[...], c_ref[...],
                        (((1,), (1,)), ((), ())),
                        preferred_element_type=jnp.float32)
    col = i * CB + lax.broadcasted_iota(jnp.int32, (1, s.shape[1]), 1)
    s = jnp.where(col >= k_items, NEG, s)
    s_ref[...] = s
    cm_ref[...] = jnp.max(s.reshape(s.shape[0], CB // CHUNK, CHUNK),
                          axis=2)[None]


def _scores_and_chunkmax(users, W, candidates):
    q, d_in = users.shape
    k_items, d = candidates.shape
    ncb = (k_items + CB - 1) // CB
    kp = ncb * CB
    cand_p = jnp.pad(candidates, ((0, kp - k_items), (0, 0)))
    return pl.pallas_call(
        functools.partial(_score_body, k_items),
        grid=(ncb,),
        in_specs=[
            pl.BlockSpec((q, d_in), lambda i: (0, 0)),
            pl.BlockSpec((d_in, d), lambda i: (0, 0)),
            pl.BlockSpec((CB, d), lambda i: (i, 0)),
        ],
        out_specs=[
            pl.BlockSpec((q, CB), lambda i: (0, i)),
            pl.BlockSpec((1, q, CB // CHUNK), lambda i: (i, 0, 0)),
        ],
        out_shape=[
            jax.ShapeDtypeStruct((q, kp), jnp.float32),
            jax.ShapeDtypeStruct((ncb, q, CB // CHUNK), jnp.float32),
        ],
        scratch_shapes=[pltpu.VMEM((q, d), jnp.float32)],
    )(users, W, cand_p)


def _thresh_body(cm_ref, thr_ref):
    def step(_, c):
        m = jnp.max(c, axis=1, keepdims=True)
        return jnp.where(c >= m, NEG, c)

    cm = lax.fori_loop(0, TOPK - 1, step, cm_ref[...])
    m = jnp.max(cm, axis=1, keepdims=True)
    thr_ref[...] = jnp.broadcast_to(m, (m.shape[0], 16))


def _thresholds(cm):
    q = cm.shape[0]
    return pl.pallas_call(
        _thresh_body,
        out_shape=jax.ShapeDtypeStruct((q, 16), jnp.float32),
    )(cm)


def _sc_body(nq, nchunk, s3, cmf, ids, vals_o, gidx_o, idn_o,
             cm_v, cid_v, ridx_v, rows_v, val_v, gix_v, idn_v, cnt_v,
             sem):
    c = lax.axis_index("c")
    s = lax.axis_index("s")
    wid = s * NC + c
    qbase = wid * nq

    def per_query(qi, carry):
        q = qbase + qi
        pltpu.sync_copy(cmf.at[pl.ds(q * (nchunk + 16), nchunk + 16)], cm_v)
        t = cm_v[pl.ds(nchunk, 16)]

        for j in range(NCID // 16 + 1):
            cid_v[pl.ds(j * 16, 16)] = jnp.full((16,), nchunk - 1, jnp.int32)
        base = jnp.zeros((16,), jnp.int32)
        for j in range(nchunk // 16):
            cm16 = cm_v[pl.ds(j * 16, 16)]
            mask = cm16 >= t
            pos = base + plsc.cumsum(_v16(1), mask=mask) - _v16(1)
            pos = jnp.minimum(jnp.maximum(pos, _v16(0)), _v16(NCID - 1))
            cids = _v16(j * 16) + lax.iota(jnp.int32, 16)
            plsc.store_scatter(cid_v, [pos], cids, mask=mask)
            base = base + plsc.all_reduce_population_count(mask)

        cnt_v[pl.ds(0, 16)] = base
        n = jnp.minimum(cnt_v[pl.ds(0, 16)][0], NCID)

        for j in range(NCID // 16):
            ridx_v[pl.ds(j * 16, 16)] = (cid_v[pl.ds(j * 16, 16)]
                                         + _v16(q * nchunk))
        pltpu.async_copy(s3.at[ridx_v], rows_v, sem).wait()

        for j in range(CAP // 16):
            val_v[pl.ds(j * 16, 16)] = jnp.full((16,), NEG, jnp.float32)
            gix_v[pl.ds(j * 16, 16)] = jnp.zeros((16,), jnp.int32)

        def per_row(r, b):
            gb = _v16(cid_v[pl.ds(r, 16)][0] * CHUNK)
            for j in range(CHUNK // 16):
                rv = rows_v[r, pl.ds(j * 16, 16)]
                mask = rv >= t
                pos = b + plsc.cumsum(_v16(1), mask=mask) - _v16(1)
                pos = jnp.minimum(jnp.maximum(pos, _v16(0)), _v16(CAP - 1))
                g = gb + _v16(j * 16) + lax.iota(jnp.int32, 16)
                plsc.store_scatter(val_v, [pos], rv, mask=mask)
                plsc.store_scatter(gix_v, [pos], g, mask=mask)
                b = b + plsc.all_reduce_population_count(mask)
            return b

        lax.fori_loop(0, n, per_row, jnp.zeros((16,), jnp.int32))

        pltpu.async_copy(ids.at[gix_v.at[pl.ds(0, 128)]],
                         idn_v.at[pl.ds(0, 128)], sem).wait()
        pltpu.async_copy(ids.at[gix_v.at[pl.ds(128, 128)]],
                         idn_v.at[pl.ds(128, 128)], sem).wait()

        pltpu.sync_copy(val_v, vals_o.at[pl.ds(q * CAP, CAP)])
        pltpu.sync_copy(gix_v, gidx_o.at[pl.ds(q * CAP, CAP)])
        pltpu.sync_copy(idn_v, idn_o.at[pl.ds(q * CAP, CAP)])
        return carry

    lax.fori_loop(0, nq, per_query, 0)


def _sc_compact(scores, cm, thr, identifiers):
    q, kp = scores.shape
    nchunk = kp // CHUNK
    assert (nchunk + 16) % 8 == 0
    nq = q // NW
    s3 = scores.reshape(q * nchunk, CHUNK)
    mesh = plsc.VectorSubcoreMesh(core_axis_name="c", subcore_axis_name="s")
    f = pl.kernel(
        functools.partial(_sc_body, nq, nchunk),
        mesh=mesh,
        compiler_params=pltpu.CompilerParams(needs_layout_passes=False),
        out_type=[
            jax.ShapeDtypeStruct((q * CAP,), jnp.float32),
            jax.ShapeDtypeStruct((q * CAP,), jnp.int32),
            jax.ShapeDtypeStruct((q * CAP,), jnp.int32),
        ],
        scratch_types=[
            pltpu.VMEM((nchunk + 16,), jnp.float32),
            pltpu.VMEM((NCID + 16,), jnp.int32),
            pltpu.VMEM((NCID,), jnp.int32),
            pltpu.VMEM((NCID, CHUNK), jnp.float32),
            pltpu.VMEM((CAP,), jnp.float32),
            pltpu.VMEM((CAP,), jnp.int32),
            pltpu.VMEM((CAP,), jnp.int32),
            pltpu.VMEM((16,), jnp.int32),
            pltpu.SemaphoreType.DMA,
        ],
    )
    cmt = jnp.concatenate([cm, thr], axis=1)
    vals, gidx, idn = f(s3, cmt.reshape(-1), identifiers)
    return (vals.reshape(q, CAP), gidx.reshape(q, CAP), idn.reshape(q, CAP))


def _select_body(v_ref, gi_ref, id_ref, out_ref):
    v = v_ref[...]
    gi = gi_ref[...]
    ident = id_ref[...]
    q = v.shape[0]
    lanes = lax.broadcasted_iota(jnp.int32, (q, 128), 1)
    intmax = jnp.int32(2**31 - 1)

    def step(k, carry):
        v, out = carry
        m = jnp.max(v, axis=1, keepdims=True)
        sel = v >= m
        chosen = jnp.min(jnp.where(sel, gi, intmax), axis=1, keepdims=True)
        hit = sel & (gi == chosen)
        cid = jnp.sum(jnp.where(hit, ident, 0), axis=1, keepdims=True)
        out = out + jnp.where(lanes == k, cid, 0)
        return jnp.where(hit, NEG, v), out

    _, out = lax.fori_loop(0, TOPK, step,
                           (v, jnp.zeros((q, 128), jnp.int32)))
    out_ref[...] = out


def _select(vals, gidx, idn):
    q = vals.shape[0]
    return pl.pallas_call(
        _select_body,
        out_shape=jax.ShapeDtypeStruct((q, 128), jnp.int32),
    )(vals, gidx, idn)


def kernel(users, W, candidates, identifiers):
    q = users.shape[0]
    scores, cm3 = _scores_and_chunkmax(users, W, candidates)
    nchunk = scores.shape[1] // CHUNK
    cm = cm3.transpose(1, 0, 2).reshape(q, nchunk)
    thr = _thresholds(cm)
    vals, gidx, idn = _sc_compact(scores, cm, thr, identifiers)
    out = _select(vals, gidx, idn)
    return out[:, :TOPK]
